# baseline (speedup 1.0000x reference)
import functools

import jax
import jax.numpy as jnp
from jax import lax
from jax.experimental import pallas as pl
from jax.experimental.pallas import tpu as pltpu
from jax.experimental.pallas import tpu_sc as plsc

_VOCAB = 100000
_EMBED = 32
_CTX = 20
_BATCH = 1024

_NC = 2
_NS = 16
_NW = _NC * _NS
_BPW = _BATCH // _NW

_LANES = 16


def _gather_sum_sc(inputs, emb_table):
    mesh = plsc.VectorSubcoreMesh(core_axis_name="c", subcore_axis_name="s")

    @functools.partial(
        pl.kernel,
        mesh=mesh,
        out_type=jax.ShapeDtypeStruct((_BATCH, _EMBED), jnp.float32),
        compiler_params=pltpu.CompilerParams(use_tc_tiling_on_sc=False),
        scratch_types=[
            pltpu.VMEM((_CTX, _BPW), jnp.int32),
            pltpu.VMEM((_CTX, _BPW, _EMBED), jnp.float32),
            pltpu.VMEM((_BPW, _EMBED), jnp.float32),
            pltpu.SemaphoreType.DMA,
        ],
    )
    def k(idx_hbm, table_hbm, out_hbm, idx_v, rows_v, acc_v, sem):
        wid = lax.axis_index("s") * _NC + lax.axis_index("c")
        base = wid * _BPW
        idx_copies = [
            pltpu.async_copy(
                idx_hbm.at[c, pl.ds(base, _BPW)], idx_v.at[c], sem
            )
            for c in range(_CTX)
        ]
        for cp in idx_copies:
            cp.wait()
        copies = [
            pltpu.async_copy(table_hbm.at[idx_v.at[c]], rows_v.at[c], sem)
            for c in range(_CTX)
        ]
        for cp in copies:
            cp.wait()

        def body(i, carry):
            for h in range(_EMBED // _LANES):
                sl = pl.ds(h * _LANES, _LANES)
                a = rows_v[0, i, sl]
                for c in range(1, _CTX):
                    a = a + rows_v[c, i, sl]
                acc_v[i, sl] = a
            return carry

        lax.fori_loop(0, _BPW, body, 0)
        pltpu.sync_copy(acc_v, out_hbm.at[pl.ds(base, _BPW)])

    return k(inputs, emb_table)


_V_BLK = 2048
_N_FULL = _VOCAB // _V_BLK
_TAIL = _VOCAB - _N_FULL * _V_BLK
_TAIL_A = (_TAIL // 128) * 128
_TAIL_B = _TAIL - _TAIL_A
_KCHUNK = 4
_ROWS = _BATCH // _KCHUNK


def _mm_body(s_ref, w_ref, b_ref, o_hbm, buf, tail_buf, sems, tail_sem):
    j = pl.program_id(0)
    slot = j % 2

    def _out_copy(sl, step, k):
        v0 = step * _V_BLK
        return pltpu.make_async_copy(
            buf.at[sl, pl.ds(k * _ROWS, _ROWS), :],
            o_hbm.at[pl.ds(k * _ROWS, _ROWS), pl.ds(v0, _V_BLK)],
            sems.at[sl, k],
        )

    def _tail_a_copy(sl, k):
        r = pl.ds(k * _ROWS, _ROWS)
        return pltpu.make_async_copy(
            buf.at[sl, r, pl.ds(0, _TAIL_A)],
            o_hbm.at[r, pl.ds(_N_FULL * _V_BLK, _TAIL_A)],
            sems.at[sl, k],
        )

    def _tail_b_copy():
        return pltpu.make_async_copy(
            tail_buf,
            o_hbm.at[:, pl.ds(_N_FULL * _V_BLK + _TAIL_A, _TAIL_B)],
            tail_sem,
        )

    @pl.when(j >= 2)
    def _():
        for k in range(_KCHUNK):
            _out_copy(slot, j - 2, k).wait()

    x = (
        lax.dot_general(
            s_ref[...], w_ref[...], (((1,), (1,)), ((), ())),
            preferred_element_type=jnp.float32,
        )
        + b_ref[...]
    )
    buf[slot] = x

    @pl.when(j < _N_FULL)
    def _():
        for k in range(_KCHUNK):
            _out_copy(slot, j, k).start()

    @pl.when(j == _N_FULL)
    def _():
        tail_buf[...] = x[:, _TAIL_A:_TAIL]
        for k in range(_KCHUNK):
            _tail_a_copy(slot, k).start()
        _tail_b_copy().start()
        for k in range(_KCHUNK):
            _out_copy(1 - slot, j - 1, k).wait()
        for k in range(_KCHUNK):
            _tail_a_copy(slot, k).wait()
        _tail_b_copy().wait()


def _project_tc(summed, W, b2d):
    return pl.pallas_call(
        _mm_body,
        grid=(_N_FULL + 1,),
        in_specs=[
            pl.BlockSpec((_BATCH, _EMBED), lambda j: (0, 0)),
            pl.BlockSpec((_V_BLK, _EMBED), lambda j: (j, 0)),
            pl.BlockSpec((1, _V_BLK), lambda j: (0, j)),
        ],
        out_specs=pl.BlockSpec(memory_space=pl.ANY),
        out_shape=jax.ShapeDtypeStruct((_BATCH, _VOCAB), jnp.float32),
        scratch_shapes=[
            pltpu.VMEM((2, _BATCH, _V_BLK), jnp.float32),
            pltpu.VMEM((_BATCH, _TAIL_B), jnp.float32),
            pltpu.SemaphoreType.DMA((2, _KCHUNK)),
            pltpu.SemaphoreType.DMA,
        ],
    )(summed, W, b2d)


def kernel(inputs, emb_table, W, b):
    summed = _gather_sum_sc(inputs.astype(jnp.int32), emb_table)
    return _project_tc(summed, W, b.reshape(1, _VOCAB))


# speedup vs baseline: 1.9671x; 1.9671x over previous
import functools

import jax
import jax.numpy as jnp
from jax import lax
from jax.experimental import pallas as pl
from jax.experimental.pallas import tpu as pltpu
from jax.experimental.pallas import tpu_sc as plsc

_VOCAB = 100000
_EMBED = 32
_CTX = 20
_BATCH = 1024

_NC = 2
_NS = 16
_NW = _NC * _NS
_BPW = _BATCH // _NW

_LANES = 16


def _gather_sum_sc(inputs, emb_table):
    mesh = plsc.VectorSubcoreMesh(core_axis_name="c", subcore_axis_name="s")

    @functools.partial(
        pl.kernel,
        mesh=mesh,
        out_type=jax.ShapeDtypeStruct((_BATCH, _EMBED), jnp.float32),
        compiler_params=pltpu.CompilerParams(use_tc_tiling_on_sc=False),
        scratch_types=[
            pltpu.VMEM((_CTX, _BPW), jnp.int32),
            pltpu.VMEM((_CTX, _BPW, _EMBED), jnp.float32),
            pltpu.VMEM((_BPW, _EMBED), jnp.float32),
            pltpu.SemaphoreType.DMA,
        ],
    )
    def k(idx_hbm, table_hbm, out_hbm, idx_v, rows_v, acc_v, sem):
        wid = lax.axis_index("s") * _NC + lax.axis_index("c")
        base = wid * _BPW
        idx_copies = [
            pltpu.async_copy(
                idx_hbm.at[c, pl.ds(base, _BPW)], idx_v.at[c], sem
            )
            for c in range(_CTX)
        ]
        for cp in idx_copies:
            cp.wait()
        copies = [
            pltpu.async_copy(table_hbm.at[idx_v.at[c]], rows_v.at[c], sem)
            for c in range(_CTX)
        ]
        for cp in copies:
            cp.wait()

        def body(i, carry):
            for h in range(_EMBED // _LANES):
                sl = pl.ds(h * _LANES, _LANES)
                a = rows_v[0, i, sl]
                for c in range(1, _CTX):
                    a = a + rows_v[c, i, sl]
                acc_v[i, sl] = a
            return carry

        lax.fori_loop(0, _BPW, body, 0)
        pltpu.sync_copy(acc_v, out_hbm.at[pl.ds(base, _BPW)])

    return k(inputs, emb_table)


_V_BLK = 2048


def _mm_body(w_ref, s_ref, b_ref, o_ref):
    o_ref[...] = (
        lax.dot_general(
            w_ref[...], s_ref[...], (((1,), (1,)), ((), ())),
            preferred_element_type=jnp.float32,
        )
        + b_ref[...]
    )


def _project_tc(summed, W, bcol):
    return pl.pallas_call(
        _mm_body,
        grid=(pl.cdiv(_VOCAB, _V_BLK),),
        in_specs=[
            pl.BlockSpec((_V_BLK, _EMBED), lambda j: (j, 0)),
            pl.BlockSpec((_BATCH, _EMBED), lambda j: (0, 0)),
            pl.BlockSpec((_V_BLK, 1), lambda j: (j, 0)),
        ],
        out_specs=pl.BlockSpec((_V_BLK, _BATCH), lambda j: (j, 0)),
        out_shape=jax.ShapeDtypeStruct((_VOCAB, _BATCH), jnp.float32),
    )(W, summed, bcol)


def kernel(inputs, emb_table, W, b):
    summed = _gather_sum_sc(inputs.astype(jnp.int32), emb_table)
    out_t = _project_tc(summed, W, b.reshape(_VOCAB, 1))
    return out_t.T


# speedup vs baseline: 2.9500x; 1.4996x over previous
import functools

import jax
import jax.numpy as jnp
from jax import lax
from jax.experimental import pallas as pl
from jax.experimental.pallas import tpu as pltpu
from jax.experimental.pallas import tpu_sc as plsc

_VOCAB = 100000
_EMBED = 32
_CTX = 20
_BATCH = 1024

_NC = 2
_NS = 16
_NW = _NC * _NS
_BPW = _BATCH // _NW

_LANES = 16


def _gather_sum_sc(inputs, emb_table):
    mesh = plsc.VectorSubcoreMesh(core_axis_name="c", subcore_axis_name="s")

    @functools.partial(
        pl.kernel,
        mesh=mesh,
        out_type=jax.ShapeDtypeStruct((_BATCH, _EMBED), jnp.float32),
        compiler_params=pltpu.CompilerParams(use_tc_tiling_on_sc=False),
        scratch_types=[
            pltpu.VMEM((_CTX, _BPW), jnp.int32),
            pltpu.VMEM((_CTX, _BPW, _EMBED), jnp.float32),
            pltpu.VMEM((_BPW, _EMBED), jnp.float32),
            pltpu.SemaphoreType.DMA,
        ],
    )
    def k(idx_hbm, table_hbm, out_hbm, idx_v, rows_v, acc_v, sem):
        wid = lax.axis_index("s") * _NC + lax.axis_index("c")
        base = wid * _BPW
        idx_copies = [
            pltpu.async_copy(
                idx_hbm.at[c, pl.ds(base, _BPW)], idx_v.at[c], sem
            )
            for c in range(_CTX)
        ]
        for cp in idx_copies:
            cp.wait()
        copies = [
            pltpu.async_copy(table_hbm.at[idx_v.at[c]], rows_v.at[c], sem)
            for c in range(_CTX)
        ]
        for cp in copies:
            cp.wait()

        def body(i, carry):
            for h in range(_EMBED // _LANES):
                sl = pl.ds(h * _LANES, _LANES)
                a = rows_v[0, i, sl]
                for c in range(1, _CTX):
                    a = a + rows_v[c, i, sl]
                acc_v[i, sl] = a
            return carry

        lax.fori_loop(0, _BPW, body, 0)
        pltpu.sync_copy(acc_v, out_hbm.at[pl.ds(base, _BPW)])

    return k(inputs, emb_table)


_V_BLK = 2048


def _mm_body(wt_ref, s_ref, b_ref, o_ref):
    prod = lax.dot_general(
        wt_ref[...], s_ref[...], (((0,), (1,)), ((), ())),
        preferred_element_type=jnp.float32,
    )
    o_ref[...] = prod + jnp.transpose(b_ref[...], (1, 0))


def _project_tc(summed, Wt, brow):
    return pl.pallas_call(
        _mm_body,
        grid=(pl.cdiv(_VOCAB, _V_BLK),),
        in_specs=[
            pl.BlockSpec((_EMBED, _V_BLK), lambda j: (0, j)),
            pl.BlockSpec((_BATCH, _EMBED), lambda j: (0, 0)),
            pl.BlockSpec((1, _V_BLK), lambda j: (0, j)),
        ],
        out_specs=pl.BlockSpec((_V_BLK, _BATCH), lambda j: (j, 0)),
        out_shape=jax.ShapeDtypeStruct((_VOCAB, _BATCH), jnp.float32),
    )(Wt, summed, brow)


def kernel(inputs, emb_table, W, b):
    summed = _gather_sum_sc(inputs.astype(jnp.int32), emb_table)
    out_t = _project_tc(summed, W.T, b.reshape(1, _VOCAB))
    return out_t.T
